# baseline (speedup 1.0000x reference)
import jax
import jax.numpy as jnp
from jax.experimental import pallas as pl
from jax.experimental.pallas import tpu as pltpu


def _cl_kernel(x_ref, w_ref, b_ref, o_ref):
    tD, H2, W2, Cin = x_ref.shape[1], x_ref.shape[2], x_ref.shape[3], x_ref.shape[4]
    tDo, Ho, Wo, Cout = o_ref.shape[1], o_ref.shape[2], o_ref.shape[3], o_ref.shape[4]
    zs = []
    for i in range(tD // 2):
        xd = x_ref[0, 2 * i] + x_ref[0, 2 * i + 1]
        x5 = xd.reshape(Ho, 2, W2, Cin)
        xh = x5[:, 0] + x5[:, 1]
        y4 = xh.reshape(Ho, Wo, 2, Cin)
        zs.append(y4[:, :, 0] + y4[:, :, 1])
    z = jnp.stack(zs, axis=0) if len(zs) > 1 else zs[0][None]
    out = jnp.dot(z.reshape(tDo * Ho * Wo, Cin), w_ref[...],
                  preferred_element_type=jnp.float32) + b_ref[...]
    o_ref[0] = out.reshape(tDo, Ho, Wo, Cout).astype(o_ref.dtype)


def _cl_path(x, w2, b2, N, Cin, Cout, D2, H2, W2, Do, Ho, Wo):
    xcl = jnp.transpose(x, (0, 2, 3, 4, 1))
    wT = jnp.transpose(w2)
    bR = b2.reshape(1, Cout)

    tD = 4 if D2 % 4 == 0 else 2
    tDo = tD // 2
    grid = (N, D2 // tD)

    in_blk = tD * H2 * W2 * Cin * 4
    out_blk = tDo * Ho * Wo * Cout * 4
    vlim = int(min(max(3 * in_blk + 3 * out_blk + (8 << 20), 32 << 20),
                   64 << 20))

    out = pl.pallas_call(
        _cl_kernel,
        out_shape=jax.ShapeDtypeStruct((N, Do, Ho, Wo, Cout), x.dtype),
        grid=grid,
        in_specs=[
            pl.BlockSpec((1, tD, H2, W2, Cin), lambda n, k: (n, k, 0, 0, 0)),
            pl.BlockSpec((Cin, Cout), lambda n, k: (0, 0)),
            pl.BlockSpec((1, Cout), lambda n, k: (0, 0)),
        ],
        out_specs=pl.BlockSpec((1, tDo, Ho, Wo, Cout),
                               lambda n, k: (n, k, 0, 0, 0)),
        compiler_params=pltpu.CompilerParams(
            dimension_semantics=("parallel", "parallel"),
            vmem_limit_bytes=vlim),
    )(xcl, wT, bR)
    return jnp.transpose(out, (0, 4, 1, 2, 3))


def _dense_kernel(x_ref, p_ref, w_ref, b_ref, o_ref):
    tD = x_ref.shape[2]
    pooled = []
    for i in range(tD // 2):
        xd = x_ref[0, :, 2 * i, :] + x_ref[0, :, 2 * i + 1, :]
        pooled.append(jnp.dot(xd, p_ref[...],
                              preferred_element_type=jnp.float32))
    z = pooled[0] if len(pooled) == 1 else jnp.concatenate(pooled, axis=1)
    out = jnp.dot(w_ref[...], z,
                  preferred_element_type=jnp.float32) + b_ref[...]
    o_ref[0] = out.astype(o_ref.dtype)


def _dense_path(x, w2, b2, N, Cin, Cout, D2, H2, W2, Do, Ho, Wo):
    HW, HoWo = H2 * W2, Ho * Wo
    x4 = x.reshape(N, Cin, D2, HW)
    hw = jnp.arange(HW)
    col = (hw // (2 * W2)) * Wo + (hw % W2) // 2
    pmat = (col[:, None] == jnp.arange(HoWo)[None, :]).astype(jnp.float32)

    tD = 8 if D2 % 8 == 0 else D2
    tDo = tD // 2
    grid = (N, D2 // tD)

    in_blk = Cin * tD * HW * 4
    out_blk = Cout * tDo * HoWo * 4
    vlim = int(min(max(3 * in_blk + 3 * out_blk + (8 << 20), 32 << 20),
                   64 << 20))

    out = pl.pallas_call(
        _dense_kernel,
        out_shape=jax.ShapeDtypeStruct((N, Cout, Do * HoWo), x.dtype),
        grid=grid,
        in_specs=[
            pl.BlockSpec((1, Cin, tD, HW), lambda n, k: (n, 0, k, 0)),
            pl.BlockSpec((HW, HoWo), lambda n, k: (0, 0)),
            pl.BlockSpec((Cout, Cin), lambda n, k: (0, 0)),
            pl.BlockSpec((Cout, 1), lambda n, k: (0, 0)),
        ],
        out_specs=pl.BlockSpec((1, Cout, tDo * HoWo), lambda n, k: (n, 0, k)),
        compiler_params=pltpu.CompilerParams(
            dimension_semantics=("parallel", "parallel"),
            vmem_limit_bytes=vlim),
    )(x4, pmat, w2, b2)
    return out.reshape(N, Cout, Do, Ho, Wo)


def kernel(x, weight, bias):
    N, Cin, D, H, W = x.shape
    Cout = weight.shape[0]
    Do, Ho, Wo = D // 2, H // 2, W // 2
    D2, H2, W2 = 2 * Do, 2 * Ho, 2 * Wo
    if (D2, H2, W2) != (D, H, W):
        x = x[:, :, :D2, :H2, :W2]

    w2 = weight.reshape(Cout, Cin).astype(jnp.float32) * 0.125
    b2 = bias.reshape(Cout, 1).astype(jnp.float32)

    if Cin % 128 == 0 and Cout % 128 == 0 and W2 % 8 == 0:
        return _cl_path(x, w2, b2, N, Cin, Cout, D2, H2, W2, Do, Ho, Wo)
    return _dense_path(x, w2, b2, N, Cin, Cout, D2, H2, W2, Do, Ho, Wo)


# speedup vs baseline: 1.4421x; 1.4421x over previous
import jax
import jax.numpy as jnp
from jax.experimental import pallas as pl
from jax.experimental.pallas import tpu as pltpu


def _cl_kernel(x_ref, w_ref, b_ref, o_ref):
    tD, H2, W2, Cin = x_ref.shape[1], x_ref.shape[2], x_ref.shape[3], x_ref.shape[4]
    tDo, Ho, Wo, Cout = o_ref.shape[1], o_ref.shape[2], o_ref.shape[3], o_ref.shape[4]
    zs = []
    for i in range(tD // 2):
        xd = x_ref[0, 2 * i] + x_ref[0, 2 * i + 1]
        x5 = xd.reshape(Ho, 2, W2, Cin)
        xh = x5[:, 0] + x5[:, 1]
        y4 = xh.reshape(Ho, Wo, 2, Cin)
        zs.append(y4[:, :, 0] + y4[:, :, 1])
    z = jnp.stack(zs, axis=0) if len(zs) > 1 else zs[0][None]
    out = jnp.dot(z.reshape(tDo * Ho * Wo, Cin), w_ref[...],
                  preferred_element_type=jnp.float32) + b_ref[...]
    o_ref[0] = out.reshape(tDo, Ho, Wo, Cout).astype(o_ref.dtype)


def _cl_path(x, w2, b2, N, Cin, Cout, D2, H2, W2, Do, Ho, Wo):
    xcl = jnp.transpose(x, (0, 2, 3, 4, 1))
    wT = jnp.transpose(w2)
    bR = b2.reshape(1, Cout)

    tD = 16 if D2 % 16 == 0 else (8 if D2 % 8 == 0 else 2)
    tDo = tD // 2
    grid = (N, D2 // tD)

    in_blk = tD * H2 * W2 * Cin * 4
    out_blk = tDo * Ho * Wo * Cout * 4
    vlim = int(min(max(3 * in_blk + 3 * out_blk + (8 << 20), 32 << 20),
                   64 << 20))

    out = pl.pallas_call(
        _cl_kernel,
        out_shape=jax.ShapeDtypeStruct((N, Do, Ho, Wo, Cout), x.dtype),
        grid=grid,
        in_specs=[
            pl.BlockSpec((1, tD, H2, W2, Cin), lambda n, k: (n, k, 0, 0, 0)),
            pl.BlockSpec((Cin, Cout), lambda n, k: (0, 0)),
            pl.BlockSpec((1, Cout), lambda n, k: (0, 0)),
        ],
        out_specs=pl.BlockSpec((1, tDo, Ho, Wo, Cout),
                               lambda n, k: (n, k, 0, 0, 0)),
        compiler_params=pltpu.CompilerParams(
            dimension_semantics=("parallel", "parallel"),
            vmem_limit_bytes=vlim),
    )(xcl, wT, bR)
    return jnp.transpose(out, (0, 4, 1, 2, 3))


def _dense_kernel(x_ref, p_ref, w_ref, b_ref, o_ref):
    tD = x_ref.shape[2]
    pooled = []
    for i in range(tD // 2):
        xd = x_ref[0, :, 2 * i, :] + x_ref[0, :, 2 * i + 1, :]
        pooled.append(jnp.dot(xd, p_ref[...],
                              preferred_element_type=jnp.float32))
    z = pooled[0] if len(pooled) == 1 else jnp.concatenate(pooled, axis=1)
    out = jnp.dot(w_ref[...], z,
                  preferred_element_type=jnp.float32) + b_ref[...]
    o_ref[0] = out.astype(o_ref.dtype)


def _dense_path(x, w2, b2, N, Cin, Cout, D2, H2, W2, Do, Ho, Wo):
    HW, HoWo = H2 * W2, Ho * Wo
    x4 = x.reshape(N, Cin, D2, HW)
    hw = jnp.arange(HW)
    col = (hw // (2 * W2)) * Wo + (hw % W2) // 2
    pmat = (col[:, None] == jnp.arange(HoWo)[None, :]).astype(jnp.float32)

    tD = 8 if D2 % 8 == 0 else D2
    tDo = tD // 2
    grid = (N, D2 // tD)

    in_blk = Cin * tD * HW * 4
    out_blk = Cout * tDo * HoWo * 4
    vlim = int(min(max(3 * in_blk + 3 * out_blk + (8 << 20), 32 << 20),
                   64 << 20))

    out = pl.pallas_call(
        _dense_kernel,
        out_shape=jax.ShapeDtypeStruct((N, Cout, Do * HoWo), x.dtype),
        grid=grid,
        in_specs=[
            pl.BlockSpec((1, Cin, tD, HW), lambda n, k: (n, 0, k, 0)),
            pl.BlockSpec((HW, HoWo), lambda n, k: (0, 0)),
            pl.BlockSpec((Cout, Cin), lambda n, k: (0, 0)),
            pl.BlockSpec((Cout, 1), lambda n, k: (0, 0)),
        ],
        out_specs=pl.BlockSpec((1, Cout, tDo * HoWo), lambda n, k: (n, 0, k)),
        compiler_params=pltpu.CompilerParams(
            dimension_semantics=("parallel", "parallel"),
            vmem_limit_bytes=vlim),
    )(x4, pmat, w2, b2)
    return out.reshape(N, Cout, Do, Ho, Wo)


def kernel(x, weight, bias):
    N, Cin, D, H, W = x.shape
    Cout = weight.shape[0]
    Do, Ho, Wo = D // 2, H // 2, W // 2
    D2, H2, W2 = 2 * Do, 2 * Ho, 2 * Wo
    if (D2, H2, W2) != (D, H, W):
        x = x[:, :, :D2, :H2, :W2]

    w2 = weight.reshape(Cout, Cin).astype(jnp.float32) * 0.125
    b2 = bias.reshape(Cout, 1).astype(jnp.float32)

    if Cin % 128 == 0 and Cout % 128 == 0 and W2 % 8 == 0:
        return _cl_path(x, w2, b2, N, Cin, Cout, D2, H2, W2, Do, Ho, Wo)
    return _dense_path(x, w2, b2, N, Cin, Cout, D2, H2, W2, Do, Ho, Wo)
